# Initial kernel scaffold; baseline (speedup 1.0000x reference)
import jax
import jax.numpy as jnp
from jax.experimental import pallas as pl


def kernel(x, edge_index, edge_attr, W1, b1, W2, b2):
    raise NotImplementedError("write your pallas kernel here")


# speedup vs baseline: 23.2090x; 23.2090x over previous
import functools

import jax
import jax.numpy as jnp
from jax import lax
from jax.experimental import pallas as pl
from jax.experimental.pallas import tpu as pltpu
from jax.experimental.pallas import tpu_sc as plsc

N = 10000
E = 320000
F_IN = 128
HID = 32
C = 40
CP = 48
N_PAD = 10240
NC = 2
NS = 16
NW = NC * NS
EPW = E // NW
CHUNK = 80
NCH = EPW // CHUNK
RPT = N_PAD // NS
LANES = 16
NG = CHUNK // LANES


def _mesh():
  return plsc.VectorSubcoreMesh(
      core_axis_name="c", subcore_axis_name="s",
      num_cores=NC, num_subcores=NS)


def _sc_params():
  return pltpu.CompilerParams(
      use_tc_tiling_on_sc=False, needs_layout_passes=False)


def _splat(buf, e):
  return plsc.load_gather(buf, [jnp.full((LANES,), e, jnp.int32)])


def _sc_deg(dst, ew):
  @functools.partial(
      pl.kernel,
      out_type=jax.ShapeDtypeStruct((NW, N_PAD), jnp.float32),
      mesh=_mesh(),
      compiler_params=_sc_params(),
      scratch_types=[
          pltpu.VMEM((EPW,), jnp.int32),
          pltpu.VMEM((EPW,), jnp.float32),
          pltpu.VMEM((N_PAD,), jnp.float32),
      ],
  )
  def k(dst_hbm, ew_hbm, out_hbm, dst_b, ew_b, acc):
    cid = lax.axis_index("c")
    sid = lax.axis_index("s")
    wid = sid * NC + cid
    zeros = jnp.zeros((LANES,), jnp.float32)

    @pl.loop(0, N_PAD // LANES)
    def _zero(i):
      acc[pl.ds(i * LANES, LANES)] = zeros

    pltpu.sync_copy(dst_hbm.at[pl.ds(wid * EPW, EPW)], dst_b)
    pltpu.sync_copy(ew_hbm.at[pl.ds(wid * EPW, EPW)], ew_b)

    @pl.loop(0, EPW // LANES)
    def _grp(g):
      iv = dst_b[pl.ds(g * LANES, LANES)]
      vv = ew_b[pl.ds(g * LANES, LANES)]
      plsc.addupdate_scatter(acc, [iv], vv)

    pltpu.sync_copy(acc, out_hbm.at[wid])

  return k(dst, ew)


def _tc1(x, W1, deg_part):
  def body(x_ref, w_ref, dp_ref, h_ref, dis_ref):
    h_ref[...] = jnp.dot(x_ref[...], w_ref[...],
                         preferred_element_type=jnp.float32)
    deg = jnp.sum(dp_ref[...], axis=0)
    dis = jnp.where(deg > 0, lax.rsqrt(jnp.where(deg > 0, deg, 1.0)), 0.0)
    dis_ref[...] = dis.reshape(N_PAD // 128, 128)

  return pl.pallas_call(
      body,
      out_shape=[
          jax.ShapeDtypeStruct((N, HID), jnp.float32),
          jax.ShapeDtypeStruct((N_PAD // 128, 128), jnp.float32),
      ],
  )(x, W1, deg_part)


def _sc_mp1(h, src2d, dst2d, ew, dis):
  @functools.partial(
      pl.kernel,
      out_type=[
          jax.ShapeDtypeStruct((NC, N_PAD, HID), jnp.float32),
          jax.ShapeDtypeStruct((E,), jnp.float32),
      ],
      mesh=_mesh(),
      compiler_params=_sc_params(),
      scratch_types=[
          pltpu.VMEM((N_PAD,), jnp.float32),
          pltpu.VMEM((NCH, CHUNK), jnp.int32),
          pltpu.VMEM((NCH, CHUNK), jnp.int32),
          pltpu.VMEM((EPW,), jnp.float32),
          pltpu.VMEM((EPW,), jnp.float32),
          pltpu.VMEM((CHUNK,), jnp.float32),
          pltpu.VMEM((CHUNK, HID), jnp.float32),
          pltpu.VMEM((CHUNK, HID), jnp.float32),
          pltpu.VMEM_SHARED((N_PAD, HID), jnp.float32),
          pltpu.SemaphoreType.DMA,
          pltpu.SemaphoreType.DMA,
      ],
  )
  def k(h_hbm, src_hbm, dst_hbm, ew_hbm, dis_hbm, out_hbm, norm_hbm,
        dis_b, src2, dst2, ew_b, norm_b, norm_c, rows0, rows1, acc,
        sem0, sem1):
    cid = lax.axis_index("c")
    sid = lax.axis_index("s")
    wid = sid * NC + cid
    pltpu.sync_copy(dis_hbm, dis_b)
    pltpu.sync_copy(src_hbm.at[pl.ds(wid * NCH, NCH)], src2)
    pltpu.sync_copy(dst_hbm.at[pl.ds(wid * NCH, NCH)], dst2)
    pltpu.sync_copy(ew_hbm.at[pl.ds(wid * EPW, EPW)], ew_b)
    zeros = jnp.zeros((LANES,), jnp.float32)

    @pl.loop(0, CHUNK)
    def _zrow(i):
      for j in range(HID // LANES):
        rows0[i, pl.ds(j * LANES, LANES)] = zeros

    @pl.loop(0, RPT // CHUNK)
    def _zacc(j):
      pltpu.sync_copy(rows0, acc.at[pl.ds(sid * RPT + j * CHUNK, CHUNK)])

    plsc.subcore_barrier()

    def gather(ci, rows, sem):
      pltpu.async_copy(h_hbm.at[src2.at[ci]], rows, sem)

    def wait(ci, rows, sem):
      pltpu.make_async_copy(h_hbm.at[src2.at[ci]], rows, sem).wait()

    def work(ci, rows):
      ebase = ci * CHUNK
      for g in range(NG):
        sv = src2[ci, pl.ds(g * LANES, LANES)]
        dv = dst2[ci, pl.ds(g * LANES, LANES)]
        wv = ew_b[pl.ds(ebase + g * LANES, LANES)]
        nv = (plsc.load_gather(dis_b, [sv]) * wv *
              plsc.load_gather(dis_b, [dv]))
        norm_c[pl.ds(g * LANES, LANES)] = nv
        norm_b[pl.ds(ebase + g * LANES, LANES)] = nv
        for i in range(LANES):
          e = g * LANES + i
          sp = _splat(norm_c, e)
          for j in range(HID // LANES):
            rows[e, pl.ds(j * LANES, LANES)] = (
                rows[e, pl.ds(j * LANES, LANES)] * sp)
      pltpu.sync_copy(rows, acc.at[dst2.at[ci]], add=True)

    gather(0, rows0, sem0)

    @pl.loop(0, (NCH - 1) // 2)
    def _pipe(j):
      ci0 = 2 * j
      gather(ci0 + 1, rows1, sem1)
      wait(ci0, rows0, sem0)
      work(ci0, rows0)
      gather(ci0 + 2, rows0, sem0)
      wait(ci0 + 1, rows1, sem1)
      work(ci0 + 1, rows1)

    wait(NCH - 1, rows0, sem0)
    work(NCH - 1, rows0)

    plsc.subcore_barrier()
    pltpu.sync_copy(norm_b, norm_hbm.at[pl.ds(wid * EPW, EPW)])
    pltpu.sync_copy(acc.at[pl.ds(sid * RPT, RPT)],
                    out_hbm.at[cid, pl.ds(sid * RPT, RPT)])

  return k(h, src2d, dst2d, ew, dis)


def _tc2(part1, b1, W2p):
  def body(p_ref, b_ref, w_ref, o_ref):
    s = p_ref[0] + p_ref[1] + b_ref[...]
    a = jnp.maximum(s, 0.0)
    o_ref[...] = jnp.dot(a, w_ref[...], preferred_element_type=jnp.float32)

  return pl.pallas_call(
      body,
      out_shape=jax.ShapeDtypeStruct((N_PAD, CP), jnp.float32),
  )(part1, b1, W2p)


def _sc_mp2(h2, src2d, dst2d, norm):
  @functools.partial(
      pl.kernel,
      out_type=jax.ShapeDtypeStruct((NC, N_PAD, CP), jnp.float32),
      mesh=_mesh(),
      compiler_params=_sc_params(),
      scratch_types=[
          pltpu.VMEM((NCH, CHUNK), jnp.int32),
          pltpu.VMEM((NCH, CHUNK), jnp.int32),
          pltpu.VMEM((EPW,), jnp.float32),
          pltpu.VMEM((CHUNK, CP), jnp.float32),
          pltpu.VMEM((CHUNK, CP), jnp.float32),
          pltpu.VMEM_SHARED((N_PAD, CP), jnp.float32),
          pltpu.SemaphoreType.DMA,
          pltpu.SemaphoreType.DMA,
      ],
  )
  def k(h_hbm, src_hbm, dst_hbm, norm_hbm, out_hbm,
        src2, dst2, norm_b, rows0, rows1, acc, sem0, sem1):
    cid = lax.axis_index("c")
    sid = lax.axis_index("s")
    wid = sid * NC + cid
    pltpu.sync_copy(src_hbm.at[pl.ds(wid * NCH, NCH)], src2)
    pltpu.sync_copy(dst_hbm.at[pl.ds(wid * NCH, NCH)], dst2)
    pltpu.sync_copy(norm_hbm.at[pl.ds(wid * EPW, EPW)], norm_b)
    zeros = jnp.zeros((LANES,), jnp.float32)

    @pl.loop(0, CHUNK)
    def _zrow(i):
      for j in range(CP // LANES):
        rows0[i, pl.ds(j * LANES, LANES)] = zeros

    @pl.loop(0, RPT // CHUNK)
    def _zacc(j):
      pltpu.sync_copy(rows0, acc.at[pl.ds(sid * RPT + j * CHUNK, CHUNK)])

    plsc.subcore_barrier()

    def gather(ci, rows, sem):
      pltpu.async_copy(h_hbm.at[src2.at[ci]], rows, sem)

    def wait(ci, rows, sem):
      pltpu.make_async_copy(h_hbm.at[src2.at[ci]], rows, sem).wait()

    def work(ci, rows):
      ebase = ci * CHUNK
      for e in range(CHUNK):
        sp = _splat(norm_b, ebase + e)
        for j in range(CP // LANES):
          rows[e, pl.ds(j * LANES, LANES)] = (
              rows[e, pl.ds(j * LANES, LANES)] * sp)
      pltpu.sync_copy(rows, acc.at[dst2.at[ci]], add=True)

    gather(0, rows0, sem0)

    @pl.loop(0, (NCH - 1) // 2)
    def _pipe(j):
      ci0 = 2 * j
      gather(ci0 + 1, rows1, sem1)
      wait(ci0, rows0, sem0)
      work(ci0, rows0)
      gather(ci0 + 2, rows0, sem0)
      wait(ci0 + 1, rows1, sem1)
      work(ci0 + 1, rows1)

    wait(NCH - 1, rows0, sem0)
    work(NCH - 1, rows0)

    plsc.subcore_barrier()
    pltpu.sync_copy(acc.at[pl.ds(sid * RPT, RPT)],
                    out_hbm.at[cid, pl.ds(sid * RPT, RPT)])

  return k(h2, src2d, dst2d, norm)


def _tc3(part2, b2p):
  def body(p_ref, b_ref, o_ref):
    s = p_ref[0] + p_ref[1] + b_ref[...]
    col = lax.broadcasted_iota(jnp.int32, (N_PAD, CP), 1)
    m = col < C
    v = jnp.where(m, s, -1e30)
    mx = jnp.max(v, axis=1, keepdims=True)
    ex = jnp.where(m, jnp.exp(v - mx), 0.0)
    lse = jnp.log(jnp.sum(ex, axis=1, keepdims=True))
    o_ref[...] = v - mx - lse

  return pl.pallas_call(
      body,
      out_shape=jax.ShapeDtypeStruct((N_PAD, CP), jnp.float32),
  )(part2, b2p)


def kernel(x, edge_index, edge_attr, W1, b1, W2, b2):
  src = edge_index[0].astype(jnp.int32)
  dst = edge_index[1].astype(jnp.int32)
  src2d = src.reshape(E // CHUNK, CHUNK)
  dst2d = dst.reshape(E // CHUNK, CHUNK)
  ew = edge_attr.astype(jnp.float32)

  deg_part = _sc_deg(dst, ew)
  h, dis2d = _tc1(x, W1, deg_part)
  dis = dis2d.reshape(N_PAD)
  part1, norm = _sc_mp1(h, src2d, dst2d, ew, dis)
  W2p = jnp.pad(W2, ((0, 0), (0, CP - C)))
  h2 = _tc2(part1, b1.reshape(1, HID), W2p)
  part2 = _sc_mp2(h2, src2d, dst2d, norm)
  out = _tc3(part2, jnp.pad(b2, (0, CP - C)).reshape(1, CP))
  return out[:N, :C]
